# Initial kernel scaffold; baseline (speedup 1.0000x reference)
import jax
import jax.numpy as jnp
from jax.experimental import pallas as pl


def kernel(x, W_experts, b_experts, W_gate, b_gate):
    raise NotImplementedError("write your pallas kernel here")


# speedup vs baseline: 3.2985x; 3.2985x over previous
import functools

import jax
import jax.numpy as jnp
from jax.experimental import pallas as pl
from jax.experimental.pallas import tpu as pltpu

HIDDEN = 1024
NUM_EXPERTS = 8
TOP_K = 2
N_TOKENS = 4096

BM = 1024


def _topk_masked_gates(logits):
    m = jnp.max(logits, axis=-1, keepdims=True)
    ex = jnp.exp(logits - m)
    g = ex / jnp.sum(ex, axis=-1, keepdims=True)
    ids = jax.lax.broadcasted_iota(jnp.int32, g.shape, 1)
    m1 = jnp.max(g, axis=-1, keepdims=True)
    a1 = jnp.min(jnp.where(g == m1, ids, NUM_EXPERTS), axis=-1, keepdims=True)
    g_wo1 = jnp.where(ids == a1, -jnp.inf, g)
    m2 = jnp.max(g_wo1, axis=-1, keepdims=True)
    a2 = jnp.min(jnp.where(g_wo1 == m2, ids, NUM_EXPERTS), axis=-1,
                 keepdims=True)
    keep = (ids == a1) | (ids == a2)
    return jnp.where(keep, g, 0.0)


def _moe_body(x_ref, wg_ref, bg_ref, we_ref, be_ref, out_ref, g_scr):
    e = pl.program_id(1)

    @pl.when(e == 0)
    def _gate():
        logits = jnp.dot(x_ref[...], wg_ref[...],
                         preferred_element_type=jnp.float32) + bg_ref[...]
        g_scr[...] = _topk_masked_gates(logits)
        out_ref[...] = jnp.dot(g_scr[...], be_ref[...],
                               preferred_element_type=jnp.float32)

    gm = g_scr[...]
    ids = jax.lax.broadcasted_iota(jnp.int32, gm.shape, 1)
    col = jnp.sum(jnp.where(ids == e, gm, 0.0), axis=-1, keepdims=True)
    out_ref[...] += col * jnp.dot(x_ref[...], we_ref[0],
                                  preferred_element_type=jnp.float32)


@jax.jit
def kernel(x, W_experts, b_experts, W_gate, b_gate):
    n = x.shape[0]
    grid = (n // BM, NUM_EXPERTS)
    return pl.pallas_call(
        _moe_body,
        grid=grid,
        in_specs=[
            pl.BlockSpec((BM, HIDDEN), lambda t, e: (t, 0)),
            pl.BlockSpec((HIDDEN, NUM_EXPERTS), lambda t, e: (0, 0)),
            pl.BlockSpec((1, NUM_EXPERTS), lambda t, e: (0, 0)),
            pl.BlockSpec((1, HIDDEN, HIDDEN), lambda t, e: (e, 0, 0)),
            pl.BlockSpec((NUM_EXPERTS, HIDDEN), lambda t, e: (0, 0)),
        ],
        out_specs=pl.BlockSpec((BM, HIDDEN), lambda t, e: (t, 0)),
        out_shape=jax.ShapeDtypeStruct((n, HIDDEN), jnp.float32),
        scratch_shapes=[pltpu.VMEM((BM, NUM_EXPERTS), jnp.float32)],
        compiler_params=pltpu.CompilerParams(
            dimension_semantics=("parallel", "arbitrary"),
        ),
    )(x, W_gate, b_gate.reshape(1, NUM_EXPERTS), W_experts, b_experts)
